# Initial kernel scaffold; baseline (speedup 1.0000x reference)
import jax
import jax.numpy as jnp
from jax.experimental import pallas as pl


def kernel(x, router_w, W1, b1, W2, b2):
    raise NotImplementedError("write your pallas kernel here")


# speedup vs baseline: 2.3494x; 2.3494x over previous
import jax
import jax.numpy as jnp
from jax.experimental import pallas as pl

N_TOK = 4096
M = 2048
HIDDEN = 512
NUM_EXPERTS = 8
TOP_K = 2
TB = 256


def _moe_dense_kernel(xb_ref, rw_ref, w1_ref, b1_ref, w2_ref, b2_ref, out_ref):
    xb = xb_ref[...]
    logits = jax.lax.dot_general(
        xb, rw_ref[...], (((1,), (1,)), ((), ())),
        preferred_element_type=jnp.float32)
    m1 = jnp.max(logits, axis=1, keepdims=True)
    cols = jax.lax.broadcasted_iota(jnp.int32, logits.shape, 1)
    sentinel = jnp.int32(NUM_EXPERTS)
    idx1 = jnp.min(jnp.where(logits == m1, cols, sentinel), axis=1, keepdims=True)
    masked = jnp.where(cols == idx1, -jnp.inf, logits)
    m2 = jnp.max(masked, axis=1, keepdims=True)
    idx2 = jnp.min(jnp.where(masked == m2, cols, sentinel), axis=1, keepdims=True)
    e2 = jnp.exp(m2 - m1)
    denom = 1.0 + e2
    w_top1 = 1.0 / denom
    w_top2 = e2 / denom

    acc = jnp.zeros((TB, M), jnp.float32)
    for e in range(NUM_EXPERTS):
        h = jax.lax.dot_general(
            xb, w1_ref[e], (((1,), (1,)), ((), ())),
            preferred_element_type=jnp.float32)
        h = jnp.maximum(h + b1_ref[e][None, :], 0.0)
        y = jax.lax.dot_general(
            h.astype(jnp.bfloat16), w2_ref[e], (((1,), (1,)), ((), ())),
            preferred_element_type=jnp.float32)
        y = y + b2_ref[e][None, :]
        we = (jnp.where(idx1 == e, w_top1, 0.0)
              + jnp.where(idx2 == e, w_top2, 0.0))
        acc = acc + we * y
    out_ref[...] = acc


def kernel(x, router_w, W1, b1, W2, b2):
    xb = x.astype(jnp.bfloat16)
    rwb = router_w.astype(jnp.bfloat16)
    W1b = W1.astype(jnp.bfloat16)
    W2b = W2.astype(jnp.bfloat16)
    grid = (N_TOK // TB,)
    return pl.pallas_call(
        _moe_dense_kernel,
        grid=grid,
        in_specs=[
            pl.BlockSpec((TB, M), lambda i: (i, 0)),
            pl.BlockSpec((NUM_EXPERTS, M), lambda i: (0, 0)),
            pl.BlockSpec((NUM_EXPERTS, HIDDEN, M), lambda i: (0, 0, 0)),
            pl.BlockSpec((NUM_EXPERTS, HIDDEN), lambda i: (0, 0)),
            pl.BlockSpec((NUM_EXPERTS, M, HIDDEN), lambda i: (0, 0, 0)),
            pl.BlockSpec((NUM_EXPERTS, M), lambda i: (0, 0)),
        ],
        out_specs=pl.BlockSpec((TB, M), lambda i: (i, 0)),
        out_shape=jax.ShapeDtypeStruct((N_TOK, M), jnp.float32),
    )(xb, rwb, W1b, b1, W2b, b2)
